# baseline (speedup 1.0000x reference)
import functools

import jax
import jax.numpy as jnp
from jax import lax
from jax.experimental import pallas as pl
from jax.experimental.pallas import tpu as pltpu
from jax.experimental.pallas import tpu_sc as plsc

HIDDEN = 768
LANES = 16
HB = HIDDEN // LANES
NUM_CORES = 2
NUM_SUBCORES = 16
NUM_WORKERS = NUM_CORES * NUM_SUBCORES
CHUNK = 32


def _sc_lookup(n_tokens):
    tpw = n_tokens // NUM_WORKERS
    n_chunks = tpw // CHUNK
    assert n_chunks % 2 == 0

    mesh = plsc.VectorSubcoreMesh(core_axis_name="c", subcore_axis_name="s")

    @functools.partial(
        pl.kernel,
        out_type=(
            jax.ShapeDtypeStruct((n_tokens, HIDDEN), jnp.float32),
            jax.ShapeDtypeStruct((n_tokens, HIDDEN), jnp.float32),
        ),
        mesh=mesh,
        scratch_types=[
            pltpu.VMEM((5, HIDDEN), jnp.float32),
            pltpu.VMEM((5, HIDDEN), jnp.float32),
            pltpu.VMEM((tpw + LANES,), jnp.int32),
            pltpu.VMEM((tpw + LANES,), jnp.float32),
            pltpu.VMEM((2 * CHUNK, HIDDEN), jnp.float32),
            pltpu.VMEM((2 * CHUNK, HIDDEN), jnp.float32),
            pltpu.SemaphoreType.DMA,
            pltpu.SemaphoreType.DMA,
            pltpu.SemaphoreType.DMA,
            pltpu.SemaphoreType.DMA,
        ],
    )
    def body(rp_hbm, mask_hbm, ktab_hbm, vtab_hbm, outk_hbm, outv_hbm,
             ktab, vtab, idx, msk, kbuf, vbuf, ks0, ks1, vs0, vs1):
        ksems = (ks0, ks1)
        vsems = (vs0, vs1)
        wid = lax.axis_index("s") * NUM_CORES + lax.axis_index("c")
        base = wid * tpw
        pltpu.sync_copy(ktab_hbm, ktab)
        pltpu.sync_copy(vtab_hbm, vtab)
        pltpu.sync_copy(rp_hbm.at[pl.ds(base, tpw)], idx.at[pl.ds(0, tpw)])
        pltpu.sync_copy(mask_hbm.at[pl.ds(base, tpw)],
                        msk.at[pl.ds(0, tpw)])

        def drain(par):
            pltpu.make_async_copy(
                kbuf.at[pl.ds(par * CHUNK, CHUNK)],
                outk_hbm.at[pl.ds(base, CHUNK)], ksems[par]).wait()
            pltpu.make_async_copy(
                vbuf.at[pl.ds(par * CHUNK, CHUNK)],
                outv_hbm.at[pl.ds(base, CHUNK)], vsems[par]).wait()

        def chunk_body(c, carry):
            p = lax.rem(c, 2)

            @pl.when(c >= 2)
            def _():
                @pl.when(p == 0)
                def _():
                    drain(0)

                @pl.when(p == 1)
                def _():
                    drain(1)

            @plsc.parallel_loop(0, CHUNK, unroll=4)
            def tok_body(t):
                tok = c * CHUNK + t
                s = idx[pl.ds(tok, LANES)][0]
                m = msk[pl.ds(tok, LANES)][0]
                row = p * CHUNK + t
                for k in range(HB):
                    sl = pl.ds(k * LANES, LANES)
                    kbuf[row, sl] = ktab[s, sl] * m
                    vbuf[row, sl] = vtab[s, sl] * m

            row0 = base + c * CHUNK

            @pl.when(p == 0)
            def _():
                pltpu.async_copy(kbuf.at[pl.ds(0, CHUNK)],
                                 outk_hbm.at[pl.ds(row0, CHUNK)], ks0)
                pltpu.async_copy(vbuf.at[pl.ds(0, CHUNK)],
                                 outv_hbm.at[pl.ds(row0, CHUNK)], vs0)

            @pl.when(p == 1)
            def _():
                pltpu.async_copy(kbuf.at[pl.ds(CHUNK, CHUNK)],
                                 outk_hbm.at[pl.ds(row0, CHUNK)], ks1)
                pltpu.async_copy(vbuf.at[pl.ds(CHUNK, CHUNK)],
                                 outv_hbm.at[pl.ds(row0, CHUNK)], vs1)

            return carry

        lax.fori_loop(0, n_chunks, chunk_body, 0, unroll=False)
        drain(0)
        drain(1)

    return body


def kernel(relative_positions, entity_mask, entity_pos_key_table,
           entity_pos_value_table):
    b, s = relative_positions.shape
    n = b * s
    rp = relative_positions.reshape(n).astype(jnp.int32)
    msk = entity_mask.reshape(n)
    out_k, out_v = _sc_lookup(n)(rp, msk, entity_pos_key_table,
                                 entity_pos_value_table)
    h = entity_pos_key_table.shape[1]
    return out_k.reshape(b, s, h), out_v.reshape(b, s, h)


# speedup vs baseline: 1.3163x; 1.3163x over previous
import functools

import jax
import jax.numpy as jnp
from jax import lax
from jax.experimental import pallas as pl
from jax.experimental.pallas import tpu as pltpu
from jax.experimental.pallas import tpu_sc as plsc

HIDDEN = 768
LANES = 16
HB = HIDDEN // LANES
NUM_CORES = 2
NUM_SUBCORES = 16
NUM_WORKERS = NUM_CORES * NUM_SUBCORES
CHUNK = 16


def _sc_lookup(n_tokens):
    tpw = n_tokens // NUM_WORKERS
    n_chunks = tpw // CHUNK
    assert n_chunks % 2 == 0

    mesh = plsc.VectorSubcoreMesh(core_axis_name="c", subcore_axis_name="s")

    @functools.partial(
        pl.kernel,
        out_type=(
            jax.ShapeDtypeStruct((n_tokens, HIDDEN), jnp.float32),
            jax.ShapeDtypeStruct((n_tokens, HIDDEN), jnp.float32),
        ),
        mesh=mesh,
        scratch_types=[
            pltpu.VMEM((5, HIDDEN), jnp.float32),
            pltpu.VMEM((5, HIDDEN), jnp.float32),
            pltpu.VMEM((tpw + LANES,), jnp.int32),
            pltpu.VMEM((tpw + LANES,), jnp.float32),
            pltpu.VMEM((2 * CHUNK, HIDDEN), jnp.float32),
            pltpu.VMEM((2 * CHUNK, HIDDEN), jnp.float32),
            pltpu.SemaphoreType.DMA,
            pltpu.SemaphoreType.DMA,
            pltpu.SemaphoreType.DMA,
            pltpu.SemaphoreType.DMA,
        ],
    )
    def body(rp_hbm, mask_hbm, ktab_hbm, vtab_hbm, outk_hbm, outv_hbm,
             ktab, vtab, idx, msk, kbuf, vbuf, ks0, ks1, vs0, vs1):
        ksems = (ks0, ks1)
        vsems = (vs0, vs1)
        wid = lax.axis_index("s") * NUM_CORES + lax.axis_index("c")
        base = wid * tpw
        pltpu.sync_copy(ktab_hbm, ktab)
        pltpu.sync_copy(vtab_hbm, vtab)
        pltpu.sync_copy(rp_hbm.at[pl.ds(base, tpw)], idx.at[pl.ds(0, tpw)])
        pltpu.sync_copy(mask_hbm.at[pl.ds(base, tpw)],
                        msk.at[pl.ds(0, tpw)])

        def drain(par):
            pltpu.make_async_copy(
                kbuf.at[pl.ds(par * CHUNK, CHUNK)],
                outk_hbm.at[pl.ds(base, CHUNK)], ksems[par]).wait()
            pltpu.make_async_copy(
                vbuf.at[pl.ds(par * CHUNK, CHUNK)],
                outv_hbm.at[pl.ds(base, CHUNK)], vsems[par]).wait()

        def chunk_body(c, carry):
            p = lax.rem(c, 2)

            @pl.when(c >= 2)
            def _():
                @pl.when(p == 0)
                def _():
                    drain(0)

                @pl.when(p == 1)
                def _():
                    drain(1)

            @plsc.parallel_loop(0, CHUNK, unroll=2)
            def tok_body(t):
                tok = c * CHUNK + t
                s = idx[pl.ds(tok, LANES)][0]
                m = msk[pl.ds(tok, LANES)][0]
                row = p * CHUNK + t
                for k in range(HB):
                    sl = pl.ds(k * LANES, LANES)
                    kbuf[row, sl] = ktab[s, sl] * m
                    vbuf[row, sl] = vtab[s, sl] * m

            row0 = base + c * CHUNK

            @pl.when(p == 0)
            def _():
                pltpu.async_copy(kbuf.at[pl.ds(0, CHUNK)],
                                 outk_hbm.at[pl.ds(row0, CHUNK)], ks0)
                pltpu.async_copy(vbuf.at[pl.ds(0, CHUNK)],
                                 outv_hbm.at[pl.ds(row0, CHUNK)], vs0)

            @pl.when(p == 1)
            def _():
                pltpu.async_copy(kbuf.at[pl.ds(CHUNK, CHUNK)],
                                 outk_hbm.at[pl.ds(row0, CHUNK)], ks1)
                pltpu.async_copy(vbuf.at[pl.ds(CHUNK, CHUNK)],
                                 outv_hbm.at[pl.ds(row0, CHUNK)], vs1)

            return carry

        lax.fori_loop(0, n_chunks, chunk_body, 0, unroll=False)
        drain(0)
        drain(1)

    return body


def kernel(relative_positions, entity_mask, entity_pos_key_table,
           entity_pos_value_table):
    b, s = relative_positions.shape
    n = b * s
    rp = relative_positions.reshape(n).astype(jnp.int32)
    msk = entity_mask.reshape(n)
    out_k, out_v = _sc_lookup(n)(rp, msk, entity_pos_key_table,
                                 entity_pos_value_table)
    h = entity_pos_key_table.shape[1]
    return out_k.reshape(b, s, h), out_v.reshape(b, s, h)


# speedup vs baseline: 2.3849x; 1.8118x over previous
import functools

import jax
import jax.numpy as jnp
from jax import lax
from jax.experimental import pallas as pl
from jax.experimental.pallas import tpu as pltpu
from jax.experimental.pallas import tpu_sc as plsc

HIDDEN = 768
LANES = 16
HB = HIDDEN // LANES
NUM_CORES = 2
NUM_SUBCORES = 16
CHUNK = 64


def _sc_lookup(n_tokens):
    tpw = n_tokens // NUM_SUBCORES
    n_chunks = tpw // CHUNK
    assert n_chunks % 2 == 0

    mesh = plsc.VectorSubcoreMesh(core_axis_name="c", subcore_axis_name="s")

    @functools.partial(
        pl.kernel,
        out_type=(
            jax.ShapeDtypeStruct((n_tokens, HIDDEN), jnp.float32),
            jax.ShapeDtypeStruct((n_tokens, HIDDEN), jnp.float32),
        ),
        mesh=mesh,
        scratch_types=[
            pltpu.VMEM((5, HIDDEN), jnp.float32),
            pltpu.VMEM((tpw + LANES,), jnp.int32),
            pltpu.VMEM((tpw + LANES,), jnp.float32),
            pltpu.VMEM((2 * CHUNK, HIDDEN), jnp.float32),
            pltpu.SemaphoreType.DMA,
            pltpu.SemaphoreType.DMA,
        ],
    )
    def body(rp_hbm, mask_hbm, tabs_hbm, outk_hbm, outv_hbm,
             tab, idx, msk, obuf, s0, s1):
        core = lax.axis_index("c")
        sub = lax.axis_index("s")
        base = sub * tpw
        pltpu.sync_copy(tabs_hbm.at[core], tab)
        pltpu.sync_copy(rp_hbm.at[pl.ds(base, tpw)], idx.at[pl.ds(0, tpw)])
        pltpu.sync_copy(mask_hbm.at[pl.ds(base, tpw)],
                        msk.at[pl.ds(0, tpw)])

        def copies(par, row0):
            return (
                pltpu.make_async_copy(
                    obuf.at[pl.ds(par * CHUNK, CHUNK)],
                    outk_hbm.at[pl.ds(row0, CHUNK)], (s0, s1)[par]),
                pltpu.make_async_copy(
                    obuf.at[pl.ds(par * CHUNK, CHUNK)],
                    outv_hbm.at[pl.ds(row0, CHUNK)], (s0, s1)[par]),
            )

        def chunk_body(c, carry):
            p = lax.rem(c, 2)

            @pl.when(c >= 2)
            def _():
                @pl.when(p == 0)
                def _():
                    copies(0, base)[0].wait()

                @pl.when(p == 1)
                def _():
                    copies(1, base)[0].wait()

            @plsc.parallel_loop(0, CHUNK, unroll=2)
            def tok_body(t):
                tok = c * CHUNK + t
                s = idx[pl.ds(tok, LANES)][0]
                m = msk[pl.ds(tok, LANES)][0]
                row = p * CHUNK + t
                for k in range(HB):
                    sl = pl.ds(k * LANES, LANES)
                    obuf[row, sl] = tab[s, sl] * m

            row0 = base + c * CHUNK
            for par in (0, 1):
                @pl.when((p == par) & (core == 0))
                def _(par=par):
                    copies(par, row0)[0].start()

                @pl.when((p == par) & (core == 1))
                def _(par=par):
                    copies(par, row0)[1].start()

            return carry

        lax.fori_loop(0, n_chunks, chunk_body, 0, unroll=False)
        copies(0, base)[0].wait()
        copies(1, base)[0].wait()

    return body


def kernel(relative_positions, entity_mask, entity_pos_key_table,
           entity_pos_value_table):
    b, s = relative_positions.shape
    n = b * s
    rp = relative_positions.reshape(n).astype(jnp.int32)
    msk = entity_mask.reshape(n)
    tabs = jnp.stack([entity_pos_key_table, entity_pos_value_table])
    out_k, out_v = _sc_lookup(n)(rp, msk, tabs)
    h = entity_pos_key_table.shape[1]
    return out_k.reshape(b, s, h), out_v.reshape(b, s, h)


# speedup vs baseline: 2.4212x; 1.0152x over previous
import functools

import jax
import jax.numpy as jnp
from jax import lax
from jax.experimental import pallas as pl
from jax.experimental.pallas import tpu as pltpu
from jax.experimental.pallas import tpu_sc as plsc

HIDDEN = 768
LANES = 16
HB = HIDDEN // LANES
NUM_CORES = 2
NUM_SUBCORES = 16
CHUNK = 64


def _sc_lookup(batch, seq):
    n_tokens = batch * seq
    tpw = n_tokens // NUM_SUBCORES
    wps = seq // tpw
    n_chunks = tpw // CHUNK
    assert n_chunks % 2 == 0 and seq % tpw == 0

    mesh = plsc.VectorSubcoreMesh(core_axis_name="c", subcore_axis_name="s")

    @functools.partial(
        pl.kernel,
        out_type=(
            jax.ShapeDtypeStruct((batch, seq, HIDDEN), jnp.float32),
            jax.ShapeDtypeStruct((batch, seq, HIDDEN), jnp.float32),
        ),
        mesh=mesh,
        scratch_types=[
            pltpu.VMEM((5, HIDDEN), jnp.float32),
            pltpu.VMEM((tpw + LANES,), jnp.int32),
            pltpu.VMEM((tpw + LANES,), jnp.float32),
            pltpu.VMEM((2 * CHUNK, HIDDEN), jnp.float32),
            pltpu.SemaphoreType.DMA,
            pltpu.SemaphoreType.DMA,
        ],
    )
    def body(rp_hbm, mask_hbm, ktab_hbm, vtab_hbm, outk_hbm, outv_hbm,
             tab, idx, msk, obuf, s0, s1):
        core = lax.axis_index("c")
        sub = lax.axis_index("s")
        bi = sub // wps
        col = (sub % wps) * tpw

        @pl.when(core == 0)
        def _():
            pltpu.sync_copy(ktab_hbm, tab)

        @pl.when(core == 1)
        def _():
            pltpu.sync_copy(vtab_hbm, tab)

        pltpu.sync_copy(rp_hbm.at[bi, pl.ds(col, tpw)],
                        idx.at[pl.ds(0, tpw)])
        pltpu.sync_copy(mask_hbm.at[bi, pl.ds(col, tpw)],
                        msk.at[pl.ds(0, tpw)])

        def copies(par, row0):
            src = obuf.at[pl.ds(par * CHUNK, CHUNK)]
            return (
                pltpu.make_async_copy(
                    src, outk_hbm.at[bi, pl.ds(row0, CHUNK)], (s0, s1)[par]),
                pltpu.make_async_copy(
                    src, outv_hbm.at[bi, pl.ds(row0, CHUNK)], (s0, s1)[par]),
            )

        def chunk_body(c, carry):
            p = lax.rem(c, 2)

            @pl.when(c >= 2)
            def _():
                @pl.when(p == 0)
                def _():
                    copies(0, col)[0].wait()

                @pl.when(p == 1)
                def _():
                    copies(1, col)[0].wait()

            @plsc.parallel_loop(0, CHUNK, unroll=2)
            def tok_body(t):
                tok = c * CHUNK + t
                s = idx[pl.ds(tok, LANES)][0]
                m = msk[pl.ds(tok, LANES)][0]
                row = p * CHUNK + t
                for k in range(HB):
                    sl = pl.ds(k * LANES, LANES)
                    obuf[row, sl] = tab[s, sl] * m

            row0 = col + c * CHUNK
            for par in (0, 1):
                @pl.when((p == par) & (core == 0))
                def _(par=par):
                    copies(par, row0)[0].start()

                @pl.when((p == par) & (core == 1))
                def _(par=par):
                    copies(par, row0)[1].start()

            return carry

        lax.fori_loop(0, n_chunks, chunk_body, 0, unroll=False)
        copies(0, col)[0].wait()
        copies(1, col)[0].wait()

    return body


def kernel(relative_positions, entity_mask, entity_pos_key_table,
           entity_pos_value_table):
    b, s = relative_positions.shape
    out_k, out_v = _sc_lookup(b, s)(
        relative_positions.astype(jnp.int32), entity_mask,
        entity_pos_key_table, entity_pos_value_table)
    return out_k, out_v
